# baseline (speedup 1.0000x reference)
import jax
import jax.numpy as jnp
from jax.experimental import pallas as pl

N = 1_000_000
PAD_N = 1_048_576
ROWS = 8192
COLS = 128
GRID = 8
BLOCK_ROWS = ROWS // GRID


def _elem_body(xg_ref, mz_ref, mn_ref, nf_ref, out_ref):
    x = xg_ref[...]
    mz = mz_ref[...] != 0
    mn = mn_ref[...] != 0
    x = jnp.where(mz, 0.0, x)
    x = jnp.where(mn, x + nf_ref[...], x)
    out_ref[...] = x


def _elementwise(xg2, mz2, mn2, nf2):
    spec = pl.BlockSpec((BLOCK_ROWS, COLS), lambda i: (i, 0))
    return pl.pallas_call(
        _elem_body,
        grid=(GRID,),
        in_specs=[spec, spec, spec, spec],
        out_specs=spec,
        out_shape=jax.ShapeDtypeStruct((ROWS, COLS), jnp.float32),
    )(xg2, mz2, mn2, nf2)


def kernel(xg, mask_zero, mask_noise, noise_full, swap_pairs):
    pad = PAD_N - N
    xg2 = jnp.concatenate([xg, jnp.zeros((pad,), jnp.float32)]).reshape(ROWS, COLS)
    mz2 = jnp.concatenate([mask_zero.astype(jnp.int8), jnp.zeros((pad,), jnp.int8)]).reshape(ROWS, COLS)
    mn2 = jnp.concatenate([mask_noise.astype(jnp.int8), jnp.zeros((pad,), jnp.int8)]).reshape(ROWS, COLS)
    nf2 = jnp.concatenate([noise_full, jnp.zeros((pad,), jnp.float32)]).reshape(ROWS, COLS)
    y = _elementwise(xg2, mz2, mn2, nf2).reshape(-1)[:N]
    old = y
    y = y.at[swap_pairs[:, 0]].set(old[swap_pairs[:, 1]])
    y = y.at[swap_pairs[:, 1]].set(old[swap_pairs[:, 0]])
    return y


# speedup vs baseline: 3.4723x; 3.4723x over previous
import functools
import jax
import jax.numpy as jnp
from jax import lax
from jax.experimental import pallas as pl
from jax.experimental.pallas import tpu as pltpu
from jax.experimental.pallas import tpu_sc as plsc

N = 1_000_000
PAD_N = 1_048_576
ROWS = 8192
COLS = 128
GRID = 8
BLOCK_ROWS = ROWS // GRID

NW = 32
OWN = PAD_N // NW
P = 50_000
PW = 2 * P
PP = 102_400
JP = PP // NW
GCH = 128
BCH = 12_800
NB = PP // BCH


def _elem_body(xg_ref, mz_ref, mn_ref, nf_ref, out_ref):
    x = xg_ref[...]
    mz = mz_ref[...] != 0
    mn = mn_ref[...] != 0
    x = jnp.where(mz, 0.0, x)
    x = jnp.where(mn, x + nf_ref[...], x)
    out_ref[...] = x


def _elementwise(xg2, mz2, mn2, nf2):
    spec = pl.BlockSpec((BLOCK_ROWS, COLS), lambda i: (i, 0))
    return pl.pallas_call(
        _elem_body,
        grid=(GRID,),
        in_specs=[spec, spec, spec, spec],
        out_specs=spec,
        out_shape=jax.ShapeDtypeStruct((ROWS, COLS), jnp.float32),
    )(xg2, mz2, mn2, nf2)


_MESH = plsc.VectorSubcoreMesh(core_axis_name="c", subcore_axis_name="s")


@functools.partial(
    pl.kernel,
    mesh=_MESH,
    out_type=jax.ShapeDtypeStruct((PP,), jnp.float32),
    scratch_types=[
        pltpu.VMEM((JP,), jnp.int32),
        pltpu.VMEM((JP,), jnp.float32),
        pltpu.SemaphoreType.DMA,
    ],
    compiler_params=pltpu.CompilerParams(needs_layout_passes=False),
)
def _gather_vals(y_hbm, src_hbm, vals_hbm, s_v, v_v, sem):
    wid = lax.axis_index("s") * 2 + lax.axis_index("c")
    base = wid * JP
    pltpu.sync_copy(src_hbm.at[pl.ds(base, JP)], s_v)

    def body(g, carry):
        off = g * GCH
        pltpu.async_copy(
            y_hbm.at[s_v.at[pl.ds(off, GCH)]],
            v_v.at[pl.ds(off, GCH)],
            sem,
        ).wait()
        return carry

    lax.fori_loop(0, JP // GCH, body, 0)
    pltpu.sync_copy(v_v, vals_hbm.at[pl.ds(base, JP)])


@functools.partial(
    pl.kernel,
    mesh=_MESH,
    out_type=jax.ShapeDtypeStruct((PAD_N,), jnp.float32),
    scratch_types=[
        pltpu.VMEM((OWN,), jnp.float32),
        pltpu.VMEM((BCH,), jnp.int32),
        pltpu.VMEM((BCH,), jnp.float32),
    ],
    compiler_params=pltpu.CompilerParams(needs_layout_passes=False),
)
def _apply_swaps(y_hbm, dest_hbm, vals_hbm, out_hbm, y_v, d_v, v_v):
    wid = lax.axis_index("s") * 2 + lax.axis_index("c")
    base = wid * OWN
    pltpu.sync_copy(y_hbm.at[pl.ds(base, OWN)], y_v)
    for c in range(NB):
        pltpu.sync_copy(dest_hbm.at[pl.ds(c * BCH, BCH)], d_v)
        pltpu.sync_copy(vals_hbm.at[pl.ds(c * BCH, BCH)], v_v)

        def ib(i, carry):
            d16 = d_v[pl.ds(i * 16, 16)] - base
            m = (d16 >= 0) & (d16 < OWN)
            dsafe = jnp.where(m, d16, 0)
            v16 = v_v[pl.ds(i * 16, 16)]
            plsc.store_scatter(y_v, [dsafe], v16, mask=m)
            return carry

        lax.fori_loop(0, BCH // 16, ib, 0)
    pltpu.sync_copy(y_v, out_hbm.at[pl.ds(base, OWN)])


def kernel(xg, mask_zero, mask_noise, noise_full, swap_pairs):
    pad = PAD_N - N
    xg2 = jnp.concatenate([xg, jnp.zeros((pad,), jnp.float32)]).reshape(ROWS, COLS)
    mz2 = jnp.concatenate([mask_zero.astype(jnp.int8), jnp.zeros((pad,), jnp.int8)]).reshape(ROWS, COLS)
    mn2 = jnp.concatenate([mask_noise.astype(jnp.int8), jnp.zeros((pad,), jnp.int8)]).reshape(ROWS, COLS)
    nf2 = jnp.concatenate([noise_full, jnp.zeros((pad,), jnp.float32)]).reshape(ROWS, COLS)
    y = _elementwise(xg2, mz2, mn2, nf2).reshape(-1)

    p0 = swap_pairs[:, 0]
    p1 = swap_pairs[:, 1]
    spad = PP - PW
    dest = jnp.concatenate([p0, p1, jnp.full((spad,), PAD_N - 1, jnp.int32)])
    src = jnp.concatenate([p1, p0, jnp.zeros((spad,), jnp.int32)])
    vals = _gather_vals(y, src)
    out = _apply_swaps(y, dest, vals)
    return out[:N]


# speedup vs baseline: 3.8043x; 1.0956x over previous
import functools
import jax
import jax.numpy as jnp
from jax import lax
from jax.experimental import pallas as pl
from jax.experimental.pallas import tpu as pltpu
from jax.experimental.pallas import tpu_sc as plsc

N = 1_000_000
PAD_N = 1_048_576
ROWS = 8192
COLS = 128
GRID = 8
BLOCK_ROWS = ROWS // GRID

NW = 32
OWN = PAD_N // NW
P = 50_000
PW = 2 * P
PP = 102_400
JP = PP // NW
GCH = 128
BCH = 12_800
NB = PP // BCH


def _elem_body(xg_ref, mz_ref, mn_ref, nf_ref, out_ref):
    x = xg_ref[...]
    mz = mz_ref[...] != 0
    mn = mn_ref[...] != 0
    x = jnp.where(mz, 0.0, x)
    x = jnp.where(mn, x + nf_ref[...], x)
    out_ref[...] = x


def _elementwise(xg2, mz2, mn2, nf2):
    spec = pl.BlockSpec((BLOCK_ROWS, COLS), lambda i: (i, 0))
    return pl.pallas_call(
        _elem_body,
        grid=(GRID,),
        in_specs=[spec, spec, spec, spec],
        out_specs=spec,
        out_shape=jax.ShapeDtypeStruct((ROWS, COLS), jnp.float32),
    )(xg2, mz2, mn2, nf2)


_MESH = plsc.VectorSubcoreMesh(core_axis_name="c", subcore_axis_name="s")


@functools.partial(
    pl.kernel,
    mesh=_MESH,
    out_type=jax.ShapeDtypeStruct((PP,), jnp.float32),
    scratch_types=[
        pltpu.VMEM((JP,), jnp.int32),
        pltpu.VMEM((JP,), jnp.float32),
        pltpu.SemaphoreType.DMA,
    ],
    compiler_params=pltpu.CompilerParams(needs_layout_passes=False),
)
def _gather_vals(y_hbm, src_hbm, vals_hbm, s_v, v_v, sem):
    wid = lax.axis_index("s") * 2 + lax.axis_index("c")
    base = wid * JP
    pltpu.sync_copy(src_hbm.at[pl.ds(base, JP)], s_v)

    copies = []
    for g in range(JP // GCH):
        off = g * GCH
        copies.append(
            pltpu.async_copy(
                y_hbm.at[s_v.at[pl.ds(off, GCH)]],
                v_v.at[pl.ds(off, GCH)],
                sem,
            )
        )
    for cp in copies:
        cp.wait()
    pltpu.sync_copy(v_v, vals_hbm.at[pl.ds(base, JP)])


@functools.partial(
    pl.kernel,
    mesh=_MESH,
    out_type=jax.ShapeDtypeStruct((PAD_N,), jnp.float32),
    scratch_types=[
        pltpu.VMEM((OWN,), jnp.float32),
        pltpu.VMEM((BCH,), jnp.int32),
        pltpu.VMEM((BCH,), jnp.float32),
    ],
    compiler_params=pltpu.CompilerParams(needs_layout_passes=False),
)
def _apply_swaps(y_hbm, dest_hbm, vals_hbm, out_hbm, y_v, d_v, v_v):
    wid = lax.axis_index("s") * 2 + lax.axis_index("c")
    base = wid * OWN
    pltpu.sync_copy(y_hbm.at[pl.ds(base, OWN)], y_v)
    for c in range(NB):
        pltpu.sync_copy(dest_hbm.at[pl.ds(c * BCH, BCH)], d_v)
        pltpu.sync_copy(vals_hbm.at[pl.ds(c * BCH, BCH)], v_v)

        def ib(i, carry):
            for u in range(4):
                d16 = d_v[pl.ds(i * 64 + u * 16, 16)] - base
                m = (d16 >= 0) & (d16 < OWN)
                dsafe = jnp.where(m, d16, 0)
                v16 = v_v[pl.ds(i * 64 + u * 16, 16)]
                plsc.store_scatter(y_v, [dsafe], v16, mask=m)
            return carry

        lax.fori_loop(0, BCH // 64, ib, 0)
    pltpu.sync_copy(y_v, out_hbm.at[pl.ds(base, OWN)])


def kernel(xg, mask_zero, mask_noise, noise_full, swap_pairs):
    pad = PAD_N - N
    xg2 = jnp.concatenate([xg, jnp.zeros((pad,), jnp.float32)]).reshape(ROWS, COLS)
    mz2 = jnp.concatenate([mask_zero.astype(jnp.int8), jnp.zeros((pad,), jnp.int8)]).reshape(ROWS, COLS)
    mn2 = jnp.concatenate([mask_noise.astype(jnp.int8), jnp.zeros((pad,), jnp.int8)]).reshape(ROWS, COLS)
    nf2 = jnp.concatenate([noise_full, jnp.zeros((pad,), jnp.float32)]).reshape(ROWS, COLS)
    y = _elementwise(xg2, mz2, mn2, nf2).reshape(-1)

    p0 = swap_pairs[:, 0]
    p1 = swap_pairs[:, 1]
    spad = PP - PW
    dest = jnp.concatenate([p0, p1, jnp.full((spad,), PAD_N - 1, jnp.int32)])
    src = jnp.concatenate([p1, p0, jnp.zeros((spad,), jnp.int32)])
    vals = _gather_vals(y, src)
    out = _apply_swaps(y, dest, vals)
    return out[:N]


# speedup vs baseline: 4.3811x; 1.1516x over previous
import functools
import jax
import jax.numpy as jnp
from jax import lax
from jax.experimental import pallas as pl
from jax.experimental.pallas import tpu as pltpu
from jax.experimental.pallas import tpu_sc as plsc

N = 1_000_000
PAD_N = 1_048_576
ROWS = 8192
COLS = 128
GRID = 8
BLOCK_ROWS = ROWS // GRID

NW = 32
OWN = PAD_N // NW
P = 50_000
PW = 2 * P
PP = 102_400
JP = PP // NW
GCH = 128
BCH = 12_800
NB = PP // BCH


def _elem_body(xg_ref, mz_ref, mn_ref, nf_ref, out_ref):
    x = xg_ref[...]
    mz = mz_ref[...] != 0
    mn = mn_ref[...] != 0
    x = jnp.where(mz, 0.0, x)
    x = jnp.where(mn, x + nf_ref[...], x)
    out_ref[...] = x


def _elementwise(xg2, mz2, mn2, nf2):
    spec = pl.BlockSpec((BLOCK_ROWS, COLS), lambda i: (i, 0))
    return pl.pallas_call(
        _elem_body,
        grid=(GRID,),
        in_specs=[spec, spec, spec, spec],
        out_specs=spec,
        out_shape=jax.ShapeDtypeStruct((ROWS, COLS), jnp.float32),
    )(xg2, mz2, mn2, nf2)


_MESH = plsc.VectorSubcoreMesh(core_axis_name="c", subcore_axis_name="s")


@functools.partial(
    pl.kernel,
    mesh=_MESH,
    out_type=jax.ShapeDtypeStruct((PP,), jnp.float32),
    scratch_types=[
        pltpu.VMEM((JP,), jnp.int32),
        pltpu.VMEM((JP,), jnp.float32),
        pltpu.SemaphoreType.DMA,
    ],
    compiler_params=pltpu.CompilerParams(needs_layout_passes=False),
)
def _gather_vals(y_hbm, src_hbm, vals_hbm, s_v, v_v, sem):
    wid = lax.axis_index("s") * 2 + lax.axis_index("c")
    base = wid * JP
    pltpu.sync_copy(src_hbm.at[pl.ds(base, JP)], s_v)

    copies = []
    for g in range(JP // GCH):
        off = g * GCH
        copies.append(
            pltpu.async_copy(
                y_hbm.at[s_v.at[pl.ds(off, GCH)]],
                v_v.at[pl.ds(off, GCH)],
                sem,
            )
        )
    for cp in copies:
        cp.wait()
    pltpu.sync_copy(v_v, vals_hbm.at[pl.ds(base, JP)])


@functools.partial(
    pl.kernel,
    mesh=_MESH,
    out_type=jax.ShapeDtypeStruct((PAD_N,), jnp.float32),
    scratch_types=[
        pltpu.VMEM((OWN,), jnp.float32),
        pltpu.VMEM((BCH,), jnp.int32),
        pltpu.VMEM((BCH,), jnp.float32),
        pltpu.VMEM((BCH,), jnp.int32),
        pltpu.VMEM((BCH,), jnp.float32),
        pltpu.SemaphoreType.DMA,
        pltpu.SemaphoreType.DMA,
    ],
    compiler_params=pltpu.CompilerParams(needs_layout_passes=False),
)
def _apply_swaps(y_hbm, dest_hbm, vals_hbm, out_hbm, y_v, d0, v0, d1, v1, semd, semv):
    wid = lax.axis_index("s") * 2 + lax.axis_index("c")
    base = wid * OWN
    dbufs = [d0, d1]
    vbufs = [v0, v1]

    def start(c):
        return (
            pltpu.async_copy(dest_hbm.at[pl.ds(c * BCH, BCH)], dbufs[c % 2], semd),
            pltpu.async_copy(vals_hbm.at[pl.ds(c * BCH, BCH)], vbufs[c % 2], semv),
        )

    pending = start(0)
    pltpu.sync_copy(y_hbm.at[pl.ds(base, OWN)], y_v)
    for c in range(NB):
        pending[0].wait()
        pending[1].wait()
        if c + 1 < NB:
            pending = start(c + 1)
        d_v = dbufs[c % 2]
        v_v = vbufs[c % 2]

        def ib(i, carry):
            for u in range(8):
                off = i * 128 + u * 16
                d16 = d_v[pl.ds(off, 16)] - base
                m = (d16 >= 0) & (d16 < OWN)
                dsafe = jnp.where(m, d16, 0)
                v16 = v_v[pl.ds(off, 16)]
                plsc.store_scatter(y_v, [dsafe], v16, mask=m)
            return carry

        lax.fori_loop(0, BCH // 128, ib, 0)
    pltpu.sync_copy(y_v, out_hbm.at[pl.ds(base, OWN)])


def kernel(xg, mask_zero, mask_noise, noise_full, swap_pairs):
    pad = PAD_N - N
    xg2 = jnp.concatenate([xg, jnp.zeros((pad,), jnp.float32)]).reshape(ROWS, COLS)
    mz2 = jnp.concatenate([mask_zero.astype(jnp.int8), jnp.zeros((pad,), jnp.int8)]).reshape(ROWS, COLS)
    mn2 = jnp.concatenate([mask_noise.astype(jnp.int8), jnp.zeros((pad,), jnp.int8)]).reshape(ROWS, COLS)
    nf2 = jnp.concatenate([noise_full, jnp.zeros((pad,), jnp.float32)]).reshape(ROWS, COLS)
    y = _elementwise(xg2, mz2, mn2, nf2).reshape(-1)

    p0 = swap_pairs[:, 0]
    p1 = swap_pairs[:, 1]
    spad = PP - PW
    dest = jnp.concatenate([p0, p1, jnp.full((spad,), PAD_N - 1, jnp.int32)])
    src = jnp.concatenate([p1, p0, jnp.zeros((spad,), jnp.int32)])
    vals = _gather_vals(y, src)
    out = _apply_swaps(y, dest, vals)
    return out[:N]
